# baseline (speedup 1.0000x reference)
import functools

import jax
import jax.numpy as jnp
from jax.experimental import pallas as pl
from jax.experimental.pallas import tpu as pltpu
from jax.experimental.pallas import tpu_sc as plsc

_K = 20
_EPS = 1e-5
_RB = 256
_PB = 2048
_NEG = float("-inf")

_NC, _NS, _L = 2, 16, 16
_W = _NC * _NS
_CH = 4
_CAP = 4096 + 16


def _score_body(xf_ref, yf_ref, xr_ref, s_ref):
    xf = xf_ref[0]
    colb = -jnp.sum(xf * xf, axis=1, keepdims=True)
    inner = jax.lax.dot_general(xr_ref[...], yf_ref[0], (((1,), (1,)), ((), ())),
                                preferred_element_type=jnp.float32)
    srow = jax.lax.dot_general(
        jnp.ones((_RB, 1), jnp.float32), colb, (((1,), (1,)), ((), ())),
        preferred_element_type=jnp.float32,
        precision=jax.lax.Precision.HIGHEST)
    s_ref[...] = 2.0 * inner + srow


def _sc_row(buf, r, cand, outb):
    neg = jnp.full((_L,), _NEG, jnp.float32)
    iota = jax.lax.iota(jnp.int32, _L)

    def p1(k, carry):
        m0, m1 = carry
        v0 = buf[r, pl.ds(2 * k * _L, _L)]
        v1 = buf[r, pl.ds((2 * k + 1) * _L, _L)]
        return jnp.maximum(m0, v0), jnp.maximum(m1, v1)

    m0, m1 = plsc.parallel_loop(0, 4096 // (2 * _L), unroll=16,
                                carry=(neg, neg))(p1)

    t = jnp.minimum(jnp.min(m0), jnp.min(m1))

    ones = jnp.ones((_L,), jnp.int32)

    def p2(k, cnt):
        v = buf[r, pl.ds(k * _L, _L)]
        msk = v >= t
        pos = plsc.cumsum(ones, mask=msk)
        plsc.store_scatter(cand, [cnt + pos - 1], v, mask=msk)
        return cnt + plsc.all_reduce_population_count(msk)

    cntv = plsc.parallel_loop(0, 4096 // _L, unroll=16,
                              carry=jnp.zeros((_L,), jnp.int32))(p2)
    c = jnp.max(cntv)

    def fast():
        vs = []
        for k in range(8):
            v = cand[pl.ds(k * _L, _L)]
            vs.append(jnp.where(k * _L + iota < c, v, neg))

        o0, o1 = neg, neg
        w = vs
        for j in range(_K):
            t1 = jnp.maximum(jnp.maximum(w[0], w[1]), jnp.maximum(w[2], w[3]))
            t2 = jnp.maximum(jnp.maximum(w[4], w[5]), jnp.maximum(w[6], w[7]))
            m = jnp.max(jnp.maximum(t1, t2))
            w = [jnp.where(v == m, _NEG, v) for v in w]
            if j < _L:
                o0 = jnp.where(iota == j, m, o0)
            else:
                o1 = jnp.where(iota == j - _L, m, o1)
        return o0, o1

    def slow():
        plsc.store_scatter(cand, [c + iota], neg)
        nv = c // _L + 1

        def p3(j, carry):
            o0, o1 = carry

            def emax(k, mm):
                return jnp.maximum(mm, cand[pl.ds(k * _L, _L)])

            m = jnp.max(jax.lax.fori_loop(0, nv, emax, neg))

            def erm(k, z):
                v = cand[pl.ds(k * _L, _L)]
                cand[pl.ds(k * _L, _L)] = jnp.where(v == m, _NEG, v)
                return z

            jax.lax.fori_loop(0, nv, erm, 0)
            o0 = jnp.where(iota == j, m, o0)
            o1 = jnp.where(iota == j - _L, m, o1)
            return o0, o1

        return jax.lax.fori_loop(0, _K, p3, (neg, neg))

    o0, o1 = jax.lax.cond(c <= 8 * _L, fast, slow)
    outb[r, pl.ds(0, _L)] = o0
    outb[r, pl.ds(_L, _L)] = o1


def _sc_topk(s2d, n_rows):
    rpw = n_rows // _W
    nch = rpw // _CH
    mesh = plsc.VectorSubcoreMesh(core_axis_name="c", subcore_axis_name="s")

    @functools.partial(
        pl.kernel,
        out_type=jax.ShapeDtypeStruct((n_rows, 2 * _L), jnp.float32),
        mesh=mesh,
        scratch_types=[
            pltpu.VMEM((_CH, 4096), jnp.float32),
            pltpu.VMEM((_CH, 4096), jnp.float32),
            pltpu.VMEM((_CAP,), jnp.float32),
            pltpu.VMEM((_CH, 2 * _L), jnp.float32),
            pltpu.SemaphoreType.DMA,
            pltpu.SemaphoreType.DMA,
        ],
        compiler_params=pltpu.CompilerParams(needs_layout_passes=False),
    )
    def topk_kernel(s_hbm, o_hbm, buf0, buf1, cand, outb, sem0, sem1):
        wid = jax.lax.axis_index("s") * _NC + jax.lax.axis_index("c")
        base = wid * rpw
        pltpu.async_copy(s_hbm.at[pl.ds(base, _CH)], buf0, sem0)

        def do_chunk(ci, buf, sem):
            pltpu.make_async_copy(
                s_hbm.at[pl.ds(base + ci * _CH, _CH)], buf, sem).wait()
            for r in range(_CH):
                _sc_row(buf, r, cand, outb)
            pltpu.sync_copy(outb, o_hbm.at[pl.ds(base + ci * _CH, _CH)])

        def pair(g, z):
            c0 = 2 * g
            pltpu.async_copy(
                s_hbm.at[pl.ds(base + (c0 + 1) * _CH, _CH)], buf1, sem1)
            do_chunk(c0, buf0, sem0)

            @pl.when(c0 + 2 < nch)
            def _():
                pltpu.async_copy(
                    s_hbm.at[pl.ds(base + (c0 + 2) * _CH, _CH)], buf0, sem0)

            do_chunk(c0 + 1, buf1, sem1)
            return z

        jax.lax.fori_loop(0, nch // 2, pair, 0)

    return topk_kernel(s2d)


def _h1_body(f_ref, y_ref, w1_ref, h1_ref, st_ref):
    rowb = -jnp.sum(y_ref[...] * y_ref[...], axis=1, keepdims=True)
    feat = f_ref[:, :_K] + rowb
    h1 = jax.lax.dot_general(feat, w1_ref[...], (((1,), (1,)), ((), ())),
                             preferred_element_type=jnp.float32)
    h1_ref[...] = h1
    s1 = jnp.sum(h1, axis=0, keepdims=True)
    s2 = jnp.sum(h1 * h1, axis=0, keepdims=True)
    st = jnp.concatenate([s1, s2], axis=0)

    @pl.when(pl.program_id(0) == 0)
    def _():
        st_ref[...] = st

    @pl.when(pl.program_id(0) != 0)
    def _():
        st_ref[...] = st_ref[...] + st


def _bn_relu_mm(h_ref, st_ref, g_ref, b_ref, w_ref, n_pos):
    s1 = st_ref[0:1, :]
    s2 = st_ref[1:2, :]
    mean = s1 / n_pos
    var = s2 / n_pos - mean * mean
    scale = g_ref[...] * jax.lax.rsqrt(var + _EPS)
    shift = b_ref[...] - mean * scale
    a = jnp.maximum(h_ref[...] * scale + shift, 0.0)
    return jax.lax.dot_general(a, w_ref[...], (((1,), (1,)), ((), ())),
                               preferred_element_type=jnp.float32)


def _k2_body(h1_ref, st_ref, g_ref, b_ref, w2_ref, h2_ref, st2_ref, *, n_pos):
    h2 = _bn_relu_mm(h1_ref, st_ref, g_ref, b_ref, w2_ref, n_pos)
    h2_ref[...] = h2
    s1 = jnp.sum(h2, axis=0, keepdims=True)
    s2 = jnp.sum(h2 * h2, axis=0, keepdims=True)
    st = jnp.concatenate([s1, s2], axis=0)

    @pl.when(pl.program_id(0) == 0)
    def _():
        st2_ref[...] = st

    @pl.when(pl.program_id(0) != 0)
    def _():
        st2_ref[...] = st2_ref[...] + st


def _k3_body(h2_ref, st_ref, g_ref, b_ref, w3_ref, o_ref, *, n_pos):
    h3 = _bn_relu_mm(h2_ref, st_ref, g_ref, b_ref, w3_ref, n_pos)
    o_ref[...] = jax.nn.sigmoid(h3)


@jax.jit
def kernel(x, y, W1, W2, W3, g1, b1, g2, b2):
    B, N, C = x.shape
    M = B * N
    n_pos = float(M)
    x2 = x.reshape(M, C)
    y2 = y.reshape(M, C)

    s2d = pl.pallas_call(
        _score_body,
        grid=(M // _RB,),
        in_specs=[
            pl.BlockSpec((1, N, C), lambda g: (g // (4096 // _RB), 0, 0)),
            pl.BlockSpec((1, N, C), lambda g: (g // (4096 // _RB), 0, 0)),
            pl.BlockSpec((_RB, C), lambda g: (g, 0)),
        ],
        out_specs=pl.BlockSpec((_RB, N), lambda g: (g, 0)),
        out_shape=jax.ShapeDtypeStruct((M, N), jnp.float32),
    )(x, y, x2)

    feat = _sc_topk(s2d, M)

    h1, st1 = pl.pallas_call(
        _h1_body,
        grid=(M // _PB,),
        in_specs=[
            pl.BlockSpec((_PB, 2 * _L), lambda i: (i, 0)),
            pl.BlockSpec((_PB, C), lambda i: (i, 0)),
            pl.BlockSpec((256, _K), lambda i: (0, 0)),
        ],
        out_specs=[
            pl.BlockSpec((_PB, 256), lambda i: (i, 0)),
            pl.BlockSpec((2, 256), lambda i: (0, 0)),
        ],
        out_shape=[
            jax.ShapeDtypeStruct((M, 256), jnp.float32),
            jax.ShapeDtypeStruct((2, 256), jnp.float32),
        ],
    )(feat, y2, W1)

    h2, st2 = pl.pallas_call(
        functools.partial(_k2_body, n_pos=n_pos),
        grid=(M // _PB,),
        in_specs=[
            pl.BlockSpec((_PB, 256), lambda i: (i, 0)),
            pl.BlockSpec((2, 256), lambda i: (0, 0)),
            pl.BlockSpec((1, 256), lambda i: (0, 0)),
            pl.BlockSpec((1, 256), lambda i: (0, 0)),
            pl.BlockSpec((128, 256), lambda i: (0, 0)),
        ],
        out_specs=[
            pl.BlockSpec((_PB, 128), lambda i: (i, 0)),
            pl.BlockSpec((2, 128), lambda i: (0, 0)),
        ],
        out_shape=[
            jax.ShapeDtypeStruct((M, 128), jnp.float32),
            jax.ShapeDtypeStruct((2, 128), jnp.float32),
        ],
    )(h1, st1, g1.reshape(1, 256), b1.reshape(1, 256), W2)

    out = pl.pallas_call(
        functools.partial(_k3_body, n_pos=n_pos),
        grid=(M // _PB,),
        in_specs=[
            pl.BlockSpec((_PB, 128), lambda i: (i, 0)),
            pl.BlockSpec((2, 128), lambda i: (0, 0)),
            pl.BlockSpec((1, 128), lambda i: (0, 0)),
            pl.BlockSpec((1, 128), lambda i: (0, 0)),
            pl.BlockSpec((1, 128), lambda i: (0, 0)),
        ],
        out_specs=pl.BlockSpec((_PB, 1), lambda i: (i, 0)),
        out_shape=jax.ShapeDtypeStruct((M, 1), jnp.float32),
    )(h2, st2, g2.reshape(1, 128), b2.reshape(1, 128), W3)

    return out.reshape(B, N, 1)


# speedup vs baseline: 1.8903x; 1.8903x over previous
import functools

import jax
import jax.numpy as jnp
from jax.experimental import pallas as pl
from jax.experimental.pallas import tpu as pltpu
from jax.experimental.pallas import tpu_sc as plsc

_K = 20
_EPS = 1e-5
_RB = 256
_PB = 2048
_NEG = float("-inf")

_NC, _NS, _L = 2, 16, 16
_W = _NC * _NS
_CH = 8
_CAP = 4096 + 16


def _score_body(xf_ref, yf_ref, xr_ref, s_ref):
    xf = xf_ref[0]
    colb = -jnp.sum(xf * xf, axis=1, keepdims=True)
    inner = jax.lax.dot_general(xr_ref[...], yf_ref[0], (((1,), (1,)), ((), ())),
                                preferred_element_type=jnp.float32)
    srow = jax.lax.dot_general(
        jnp.ones((_RB, 1), jnp.float32), colb, (((1,), (1,)), ((), ())),
        preferred_element_type=jnp.float32,
        precision=jax.lax.Precision.HIGHEST)
    s_ref[...] = 2.0 * inner + srow


def _tc_topk_body(xf_ref, yf_ref, xr_ref, yr_ref, w1_ref, h1_ref, st_ref):
    g = pl.program_id(0)
    xf = xf_ref[0]
    colb = -jnp.sum(xf * xf, axis=1, keepdims=True)
    inner = jax.lax.dot_general(xr_ref[...], yf_ref[0], (((1,), (1,)), ((), ())),
                                preferred_element_type=jnp.float32)
    srow = jax.lax.dot_general(
        jnp.ones((_RB, 1), jnp.float32), colb, (((1,), (1,)), ((), ())),
        preferred_element_type=jnp.float32,
        precision=jax.lax.Precision.HIGHEST)
    s = 2.0 * inner + srow

    out = jnp.full((_RB, 32), _NEG, jnp.float32)
    lane = jax.lax.broadcasted_iota(jnp.int32, (_RB, 32), 1)
    for j in range(_K):
        m = jnp.max(s, axis=1, keepdims=True)
        s = jnp.where(s == m, _NEG, s)
        out = jnp.where(lane == j, m, out)

    rowb = -jnp.sum(yr_ref[...] * yr_ref[...], axis=1, keepdims=True)
    feat = out[:, :_K] + rowb
    h1 = jax.lax.dot_general(feat, w1_ref[...], (((1,), (1,)), ((), ())),
                             preferred_element_type=jnp.float32)
    h1_ref[...] = h1
    s1 = jnp.sum(h1, axis=0, keepdims=True)
    s2 = jnp.sum(h1 * h1, axis=0, keepdims=True)
    st = jnp.concatenate([s1, s2], axis=0)

    @pl.when(g == 0)
    def _():
        st_ref[...] = st

    @pl.when(g != 0)
    def _():
        st_ref[...] = st_ref[...] + st


def _sc_row(buf, r, cand, outb):
    neg = jnp.full((_L,), _NEG, jnp.float32)
    iota = jax.lax.iota(jnp.int32, _L)

    def p1(k, carry):
        m0, m1 = carry
        v0 = buf[r, pl.ds(2 * k * _L, _L)]
        v1 = buf[r, pl.ds((2 * k + 1) * _L, _L)]
        return jnp.maximum(m0, v0), jnp.maximum(m1, v1)

    m0, m1 = plsc.parallel_loop(0, 4096 // (2 * _L), unroll=8,
                                carry=(neg, neg))(p1)

    t = jnp.minimum(jnp.min(m0), jnp.min(m1))

    ones = jnp.ones((_L,), jnp.int32)

    def p2(k, cnt):
        v = buf[r, pl.ds(k * _L, _L)]
        msk = v >= t
        pos = plsc.cumsum(ones, mask=msk)
        plsc.store_scatter(cand, [cnt + pos - 1], v, mask=msk)
        return cnt + plsc.all_reduce_population_count(msk)

    cntv = plsc.parallel_loop(0, 4096 // _L, unroll=8,
                              carry=jnp.zeros((_L,), jnp.int32))(p2)
    c = jnp.max(cntv)

    def fast():
        vs = []
        for k in range(8):
            v = cand[pl.ds(k * _L, _L)]
            vs.append(jnp.where(k * _L + iota < c, v, neg))

        def rnd(j, carry):
            o0, o1 = carry[0], carry[1]
            w = list(carry[2:])
            t1 = jnp.maximum(jnp.maximum(w[0], w[1]), jnp.maximum(w[2], w[3]))
            t2 = jnp.maximum(jnp.maximum(w[4], w[5]), jnp.maximum(w[6], w[7]))
            m = jnp.max(jnp.maximum(t1, t2))
            w = [jnp.where(v == m, _NEG, v) for v in w]
            o0 = jnp.where(iota == j, m, o0)
            o1 = jnp.where(iota == j - _L, m, o1)
            return (o0, o1, *w)

        res = jax.lax.fori_loop(0, _K, rnd, (neg, neg, *vs))
        return res[0], res[1]

    def slow():
        plsc.store_scatter(cand, [c + iota], neg)
        nv = c // _L + 1

        def p3(j, carry):
            o0, o1 = carry

            def emax(k, mm):
                return jnp.maximum(mm, cand[pl.ds(k * _L, _L)])

            m = jnp.max(jax.lax.fori_loop(0, nv, emax, neg))

            def erm(k, z):
                v = cand[pl.ds(k * _L, _L)]
                cand[pl.ds(k * _L, _L)] = jnp.where(v == m, _NEG, v)
                return z

            jax.lax.fori_loop(0, nv, erm, 0)
            o0 = jnp.where(iota == j, m, o0)
            o1 = jnp.where(iota == j - _L, m, o1)
            return o0, o1

        return jax.lax.fori_loop(0, _K, p3, (neg, neg))

    o0, o1 = jax.lax.cond(c <= 8 * _L, fast, slow)
    outb[r, pl.ds(0, _L)] = o0
    outb[r, pl.ds(_L, _L)] = o1


def _sc_topk(s2d, n_rows):
    rpw = n_rows // _W
    nch = rpw // _CH
    mesh = plsc.VectorSubcoreMesh(core_axis_name="c", subcore_axis_name="s")

    @functools.partial(
        pl.kernel,
        out_type=jax.ShapeDtypeStruct((n_rows, 2 * _L), jnp.float32),
        mesh=mesh,
        scratch_types=[
            pltpu.VMEM((_CH, 4096), jnp.float32),
            pltpu.VMEM((_CH, 4096), jnp.float32),
            pltpu.VMEM((_CAP,), jnp.float32),
            pltpu.VMEM((_CH, 2 * _L), jnp.float32),
            pltpu.SemaphoreType.DMA,
            pltpu.SemaphoreType.DMA,
        ],
        compiler_params=pltpu.CompilerParams(needs_layout_passes=False),
    )
    def topk_kernel(s_hbm, o_hbm, buf0, buf1, cand, outb, sem0, sem1):
        wid = jax.lax.axis_index("s") * _NC + jax.lax.axis_index("c")
        base = wid * rpw
        pltpu.async_copy(s_hbm.at[pl.ds(base, _CH)], buf0, sem0)

        def do_chunk(ci, buf, sem):
            pltpu.make_async_copy(
                s_hbm.at[pl.ds(base + ci * _CH, _CH)], buf, sem).wait()
            for r in range(_CH):
                _sc_row(buf, r, cand, outb)
            pltpu.sync_copy(outb, o_hbm.at[pl.ds(base + ci * _CH, _CH)])

        def pair(g, z):
            c0 = 2 * g
            pltpu.async_copy(
                s_hbm.at[pl.ds(base + (c0 + 1) * _CH, _CH)], buf1, sem1)
            do_chunk(c0, buf0, sem0)

            @pl.when(c0 + 2 < nch)
            def _():
                pltpu.async_copy(
                    s_hbm.at[pl.ds(base + (c0 + 2) * _CH, _CH)], buf0, sem0)

            do_chunk(c0 + 1, buf1, sem1)
            return z

        jax.lax.fori_loop(0, nch // 2, pair, 0)

    return topk_kernel(s2d)


def _h1_body(f_ref, y_ref, w1_ref, h1_ref, st_ref):
    rowb = -jnp.sum(y_ref[...] * y_ref[...], axis=1, keepdims=True)
    feat = f_ref[:, :_K] + rowb
    h1 = jax.lax.dot_general(feat, w1_ref[...], (((1,), (1,)), ((), ())),
                             preferred_element_type=jnp.float32)
    h1_ref[...] = h1
    s1 = jnp.sum(h1, axis=0, keepdims=True)
    s2 = jnp.sum(h1 * h1, axis=0, keepdims=True)
    st = jnp.concatenate([s1, s2], axis=0)

    @pl.when(pl.program_id(0) == 0)
    def _():
        st_ref[...] = st

    @pl.when(pl.program_id(0) != 0)
    def _():
        st_ref[...] = st_ref[...] + st


def _bn_relu_mm(h_ref, st_ref, g_ref, b_ref, w_ref, n_pos):
    s1 = st_ref[0:1, :]
    s2 = st_ref[1:2, :]
    mean = s1 / n_pos
    var = s2 / n_pos - mean * mean
    scale = g_ref[...] * jax.lax.rsqrt(var + _EPS)
    shift = b_ref[...] - mean * scale
    a = jnp.maximum(h_ref[...] * scale + shift, 0.0)
    return jax.lax.dot_general(a, w_ref[...], (((1,), (1,)), ((), ())),
                               preferred_element_type=jnp.float32)


def _k2_body(h1_ref, st_ref, g_ref, b_ref, w2_ref, h2_ref, st2_ref, *, n_pos):
    h2 = _bn_relu_mm(h1_ref, st_ref, g_ref, b_ref, w2_ref, n_pos)
    h2_ref[...] = h2
    s1 = jnp.sum(h2, axis=0, keepdims=True)
    s2 = jnp.sum(h2 * h2, axis=0, keepdims=True)
    st = jnp.concatenate([s1, s2], axis=0)

    @pl.when(pl.program_id(0) == 0)
    def _():
        st2_ref[...] = st

    @pl.when(pl.program_id(0) != 0)
    def _():
        st2_ref[...] = st2_ref[...] + st


def _k3_body(h2_ref, st_ref, g_ref, b_ref, w3_ref, o_ref, *, n_pos):
    h3 = _bn_relu_mm(h2_ref, st_ref, g_ref, b_ref, w3_ref, n_pos)
    o_ref[...] = jax.nn.sigmoid(h3)


_A = 20480


@jax.jit
def kernel(x, y, W1, W2, W3, g1, b1, g2, b2):
    B, N, C = x.shape
    M = B * N
    n_pos = float(M)
    x2 = x.reshape(M, C)
    y2 = y.reshape(M, C)
    nblk = N // _RB
    a_blk = _A // _RB
    a_pb = _A // _PB

    s2d = pl.pallas_call(
        _score_body,
        grid=((M - _A) // _RB,),
        in_specs=[
            pl.BlockSpec((1, N, C), lambda g: ((a_blk + g) // nblk, 0, 0)),
            pl.BlockSpec((1, N, C), lambda g: ((a_blk + g) // nblk, 0, 0)),
            pl.BlockSpec((_RB, C), lambda g: (a_blk + g, 0)),
        ],
        out_specs=pl.BlockSpec((_RB, N), lambda g: (g, 0)),
        out_shape=jax.ShapeDtypeStruct((M - _A, N), jnp.float32),
    )(x, y, x2)

    feat = _sc_topk(s2d, M - _A)

    h1b, stb = pl.pallas_call(
        _h1_body,
        grid=((M - _A) // _PB,),
        in_specs=[
            pl.BlockSpec((_PB, 2 * _L), lambda i: (i, 0)),
            pl.BlockSpec((_PB, C), lambda i: (a_pb + i, 0)),
            pl.BlockSpec((256, _K), lambda i: (0, 0)),
        ],
        out_specs=[
            pl.BlockSpec((_PB, 256), lambda i: (i, 0)),
            pl.BlockSpec((2, 256), lambda i: (0, 0)),
        ],
        out_shape=[
            jax.ShapeDtypeStruct((M - _A, 256), jnp.float32),
            jax.ShapeDtypeStruct((2, 256), jnp.float32),
        ],
    )(feat, y2, W1)

    h1a, sta = pl.pallas_call(
        _tc_topk_body,
        grid=(a_blk,),
        in_specs=[
            pl.BlockSpec((1, N, C), lambda g: (g // nblk, 0, 0)),
            pl.BlockSpec((1, N, C), lambda g: (g // nblk, 0, 0)),
            pl.BlockSpec((_RB, C), lambda g: (g, 0)),
            pl.BlockSpec((_RB, C), lambda g: (g, 0)),
            pl.BlockSpec((256, _K), lambda g: (0, 0)),
        ],
        out_specs=[
            pl.BlockSpec((_RB, 256), lambda g: (g, 0)),
            pl.BlockSpec((2, 256), lambda g: (0, 0)),
        ],
        out_shape=[
            jax.ShapeDtypeStruct((_A, 256), jnp.float32),
            jax.ShapeDtypeStruct((2, 256), jnp.float32),
        ],
    )(x, y, x2, y2, W1)

    h1 = jnp.concatenate([h1a, h1b], axis=0)
    st1 = sta + stb

    h2, st2 = pl.pallas_call(
        functools.partial(_k2_body, n_pos=n_pos),
        grid=(M // _PB,),
        in_specs=[
            pl.BlockSpec((_PB, 256), lambda i: (i, 0)),
            pl.BlockSpec((2, 256), lambda i: (0, 0)),
            pl.BlockSpec((1, 256), lambda i: (0, 0)),
            pl.BlockSpec((1, 256), lambda i: (0, 0)),
            pl.BlockSpec((128, 256), lambda i: (0, 0)),
        ],
        out_specs=[
            pl.BlockSpec((_PB, 128), lambda i: (i, 0)),
            pl.BlockSpec((2, 128), lambda i: (0, 0)),
        ],
        out_shape=[
            jax.ShapeDtypeStruct((M, 128), jnp.float32),
            jax.ShapeDtypeStruct((2, 128), jnp.float32),
        ],
    )(h1, st1, g1.reshape(1, 256), b1.reshape(1, 256), W2)

    out = pl.pallas_call(
        functools.partial(_k3_body, n_pos=n_pos),
        grid=(M // _PB,),
        in_specs=[
            pl.BlockSpec((_PB, 128), lambda i: (i, 0)),
            pl.BlockSpec((2, 128), lambda i: (0, 0)),
            pl.BlockSpec((1, 128), lambda i: (0, 0)),
            pl.BlockSpec((1, 128), lambda i: (0, 0)),
            pl.BlockSpec((1, 128), lambda i: (0, 0)),
        ],
        out_specs=pl.BlockSpec((_PB, 1), lambda i: (i, 0)),
        out_shape=jax.ShapeDtypeStruct((M, 1), jnp.float32),
    )(h2, st2, g2.reshape(1, 128), b2.reshape(1, 128), W3)

    return out.reshape(B, N, 1)


# speedup vs baseline: 1.9884x; 1.0519x over previous
import functools

import jax
import jax.numpy as jnp
from jax.experimental import pallas as pl
from jax.experimental.pallas import tpu as pltpu
from jax.experimental.pallas import tpu_sc as plsc

_K = 20
_EPS = 1e-5
_RB = 256
_PB = 2048
_NEG = float("-inf")

_NC, _NS, _L = 2, 16, 16
_W = _NC * _NS
_CH = 8
_CAP = 4096 + 16


def _score_body(xf_ref, yf_ref, xr_ref, s_ref):
    xf = xf_ref[0]
    colb = -jnp.sum(xf * xf, axis=1, keepdims=True)
    inner = jax.lax.dot_general(xr_ref[...], yf_ref[0], (((1,), (1,)), ((), ())),
                                preferred_element_type=jnp.float32)
    srow = jax.lax.dot_general(
        jnp.ones((_RB, 1), jnp.float32), colb, (((1,), (1,)), ((), ())),
        preferred_element_type=jnp.float32,
        precision=jax.lax.Precision.HIGHEST)
    s_ref[...] = 2.0 * inner + srow


def _tc_topk_body(xf_ref, yf_ref, xr_ref, yr_ref, w1_ref, h1_ref, st_ref):
    g = pl.program_id(0)
    xf = xf_ref[0]
    colb = -jnp.sum(xf * xf, axis=1, keepdims=True)
    inner = jax.lax.dot_general(xr_ref[...], yf_ref[0], (((1,), (1,)), ((), ())),
                                preferred_element_type=jnp.float32)
    srow = jax.lax.dot_general(
        jnp.ones((_RB, 1), jnp.float32), colb, (((1,), (1,)), ((), ())),
        preferred_element_type=jnp.float32,
        precision=jax.lax.Precision.HIGHEST)
    s = 2.0 * inner + srow

    out = jnp.full((_RB, 32), _NEG, jnp.float32)
    lane = jax.lax.broadcasted_iota(jnp.int32, (_RB, 32), 1)
    for j in range(_K):
        m = jnp.max(s, axis=1, keepdims=True)
        s = jnp.where(s == m, _NEG, s)
        out = jnp.where(lane == j, m, out)

    rowb = -jnp.sum(yr_ref[...] * yr_ref[...], axis=1, keepdims=True)
    feat = out[:, :_K] + rowb
    h1 = jax.lax.dot_general(feat, w1_ref[...], (((1,), (1,)), ((), ())),
                             preferred_element_type=jnp.float32)
    h1_ref[...] = h1
    s1 = jnp.sum(h1, axis=0, keepdims=True)
    s2 = jnp.sum(h1 * h1, axis=0, keepdims=True)
    st = jnp.concatenate([s1, s2], axis=0)

    @pl.when(g == 0)
    def _():
        st_ref[...] = st

    @pl.when(g != 0)
    def _():
        st_ref[...] = st_ref[...] + st


def _sc_row(buf, r, cand, outb):
    neg = jnp.full((_L,), _NEG, jnp.float32)
    iota = jax.lax.iota(jnp.int32, _L)

    def p1(k, carry):
        m0, m1 = carry
        v0 = buf[r, pl.ds(2 * k * _L, _L)]
        v1 = buf[r, pl.ds((2 * k + 1) * _L, _L)]
        return jnp.maximum(m0, v0), jnp.maximum(m1, v1)

    m0, m1 = plsc.parallel_loop(0, 4096 // (2 * _L), unroll=8,
                                carry=(neg, neg))(p1)

    t = jnp.minimum(jnp.min(m0), jnp.min(m1))

    ones = jnp.ones((_L,), jnp.int32)

    def p2(k, cnt):
        v = buf[r, pl.ds(k * _L, _L)]
        msk = v >= t
        pos = plsc.cumsum(ones, mask=msk)
        plsc.store_scatter(cand, [cnt + pos - 1], v, mask=msk)
        return cnt + plsc.all_reduce_population_count(msk)

    cntv = plsc.parallel_loop(0, 4096 // _L, unroll=8,
                              carry=jnp.zeros((_L,), jnp.int32))(p2)
    c = jnp.max(cntv)

    def fast():
        vs = []
        for k in range(8):
            v = cand[pl.ds(k * _L, _L)]
            vs.append(jnp.where(k * _L + iota < c, v, neg))

        def rnd(j, carry):
            o0, o1 = carry[0], carry[1]
            w = list(carry[2:])
            t1 = jnp.maximum(jnp.maximum(w[0], w[1]), jnp.maximum(w[2], w[3]))
            t2 = jnp.maximum(jnp.maximum(w[4], w[5]), jnp.maximum(w[6], w[7]))
            m = jnp.max(jnp.maximum(t1, t2))
            w = [jnp.where(v == m, _NEG, v) for v in w]
            o0 = jnp.where(iota == j, m, o0)
            o1 = jnp.where(iota == j - _L, m, o1)
            return (o0, o1, *w)

        res = jax.lax.fori_loop(0, _K, rnd, (neg, neg, *vs))
        return res[0], res[1]

    def slow():
        plsc.store_scatter(cand, [c + iota], neg)
        nv = c // _L + 1

        def p3(j, carry):
            o0, o1 = carry

            def emax(k, mm):
                return jnp.maximum(mm, cand[pl.ds(k * _L, _L)])

            m = jnp.max(jax.lax.fori_loop(0, nv, emax, neg))

            def erm(k, z):
                v = cand[pl.ds(k * _L, _L)]
                cand[pl.ds(k * _L, _L)] = jnp.where(v == m, _NEG, v)
                return z

            jax.lax.fori_loop(0, nv, erm, 0)
            o0 = jnp.where(iota == j, m, o0)
            o1 = jnp.where(iota == j - _L, m, o1)
            return o0, o1

        return jax.lax.fori_loop(0, _K, p3, (neg, neg))

    o0, o1 = jax.lax.cond(c <= 8 * _L, fast, slow)
    outb[r, pl.ds(0, _L)] = o0
    outb[r, pl.ds(_L, _L)] = o1


def _sc_topk(s2d, n_rows):
    rpw = n_rows // _W
    nch = rpw // _CH
    mesh = plsc.VectorSubcoreMesh(core_axis_name="c", subcore_axis_name="s")

    @functools.partial(
        pl.kernel,
        out_type=jax.ShapeDtypeStruct((n_rows, 2 * _L), jnp.float32),
        mesh=mesh,
        scratch_types=[
            pltpu.VMEM((_CH, 4096), jnp.float32),
            pltpu.VMEM((_CH, 4096), jnp.float32),
            pltpu.VMEM((_CAP,), jnp.float32),
            pltpu.VMEM((_CH, 2 * _L), jnp.float32),
            pltpu.SemaphoreType.DMA,
            pltpu.SemaphoreType.DMA,
        ],
        compiler_params=pltpu.CompilerParams(needs_layout_passes=False),
    )
    def topk_kernel(s_hbm, o_hbm, buf0, buf1, cand, outb, sem0, sem1):
        wid = jax.lax.axis_index("s") * _NC + jax.lax.axis_index("c")
        base = wid * rpw
        pltpu.async_copy(s_hbm.at[pl.ds(base, _CH)], buf0, sem0)

        def do_chunk(ci, buf, sem):
            pltpu.make_async_copy(
                s_hbm.at[pl.ds(base + ci * _CH, _CH)], buf, sem).wait()
            for r in range(_CH):
                _sc_row(buf, r, cand, outb)
            pltpu.sync_copy(outb, o_hbm.at[pl.ds(base + ci * _CH, _CH)])

        def pair(g, z):
            c0 = 2 * g
            pltpu.async_copy(
                s_hbm.at[pl.ds(base + (c0 + 1) * _CH, _CH)], buf1, sem1)
            do_chunk(c0, buf0, sem0)

            @pl.when(c0 + 2 < nch)
            def _():
                pltpu.async_copy(
                    s_hbm.at[pl.ds(base + (c0 + 2) * _CH, _CH)], buf0, sem0)

            do_chunk(c0 + 1, buf1, sem1)
            return z

        jax.lax.fori_loop(0, nch // 2, pair, 0)

    return topk_kernel(s2d)


def _h1_body(f_ref, y_ref, w1_ref, h1_ref, st_ref):
    rowb = -jnp.sum(y_ref[...] * y_ref[...], axis=1, keepdims=True)
    feat = f_ref[:, :_K] + rowb
    h1 = jax.lax.dot_general(feat, w1_ref[...], (((1,), (1,)), ((), ())),
                             preferred_element_type=jnp.float32)
    h1_ref[...] = h1
    s1 = jnp.sum(h1, axis=0, keepdims=True)
    s2 = jnp.sum(h1 * h1, axis=0, keepdims=True)
    st = jnp.concatenate([s1, s2], axis=0)

    @pl.when(pl.program_id(0) == 0)
    def _():
        st_ref[...] = st

    @pl.when(pl.program_id(0) != 0)
    def _():
        st_ref[...] = st_ref[...] + st


def _bn_relu_mm(h_ref, st_ref, g_ref, b_ref, w_ref, n_pos):
    s1 = st_ref[0:1, :]
    s2 = st_ref[1:2, :]
    mean = s1 / n_pos
    var = s2 / n_pos - mean * mean
    scale = g_ref[...] * jax.lax.rsqrt(var + _EPS)
    shift = b_ref[...] - mean * scale
    a = jnp.maximum(h_ref[...] * scale + shift, 0.0)
    return jax.lax.dot_general(a, w_ref[...], (((1,), (1,)), ((), ())),
                               preferred_element_type=jnp.float32)


def _k2_body(h1_ref, st_ref, g_ref, b_ref, w2_ref, h2_ref, st2_ref, *, n_pos):
    h2 = _bn_relu_mm(h1_ref, st_ref, g_ref, b_ref, w2_ref, n_pos)
    h2_ref[...] = h2
    s1 = jnp.sum(h2, axis=0, keepdims=True)
    s2 = jnp.sum(h2 * h2, axis=0, keepdims=True)
    st = jnp.concatenate([s1, s2], axis=0)

    @pl.when(pl.program_id(0) == 0)
    def _():
        st2_ref[...] = st

    @pl.when(pl.program_id(0) != 0)
    def _():
        st2_ref[...] = st2_ref[...] + st


def _k3_body(h2_ref, st_ref, g_ref, b_ref, w3_ref, o_ref, *, n_pos):
    h3 = _bn_relu_mm(h2_ref, st_ref, g_ref, b_ref, w3_ref, n_pos)
    o_ref[...] = jax.nn.sigmoid(h3)


_A = 18432


@jax.jit
def kernel(x, y, W1, W2, W3, g1, b1, g2, b2):
    B, N, C = x.shape
    M = B * N
    n_pos = float(M)
    x2 = x.reshape(M, C)
    y2 = y.reshape(M, C)
    nblk = N // _RB
    a_blk = _A // _RB
    a_pb = _A // _PB

    s2d = pl.pallas_call(
        _score_body,
        grid=((M - _A) // _RB,),
        in_specs=[
            pl.BlockSpec((1, N, C), lambda g: ((a_blk + g) // nblk, 0, 0)),
            pl.BlockSpec((1, N, C), lambda g: ((a_blk + g) // nblk, 0, 0)),
            pl.BlockSpec((_RB, C), lambda g: (a_blk + g, 0)),
        ],
        out_specs=pl.BlockSpec((_RB, N), lambda g: (g, 0)),
        out_shape=jax.ShapeDtypeStruct((M - _A, N), jnp.float32),
    )(x, y, x2)

    feat = _sc_topk(s2d, M - _A)

    h1b, stb = pl.pallas_call(
        _h1_body,
        grid=((M - _A) // _PB,),
        in_specs=[
            pl.BlockSpec((_PB, 2 * _L), lambda i: (i, 0)),
            pl.BlockSpec((_PB, C), lambda i: (a_pb + i, 0)),
            pl.BlockSpec((256, _K), lambda i: (0, 0)),
        ],
        out_specs=[
            pl.BlockSpec((_PB, 256), lambda i: (i, 0)),
            pl.BlockSpec((2, 256), lambda i: (0, 0)),
        ],
        out_shape=[
            jax.ShapeDtypeStruct((M - _A, 256), jnp.float32),
            jax.ShapeDtypeStruct((2, 256), jnp.float32),
        ],
    )(feat, y2, W1)

    h1a, sta = pl.pallas_call(
        _tc_topk_body,
        grid=(a_blk,),
        in_specs=[
            pl.BlockSpec((1, N, C), lambda g: (g // nblk, 0, 0)),
            pl.BlockSpec((1, N, C), lambda g: (g // nblk, 0, 0)),
            pl.BlockSpec((_RB, C), lambda g: (g, 0)),
            pl.BlockSpec((_RB, C), lambda g: (g, 0)),
            pl.BlockSpec((256, _K), lambda g: (0, 0)),
        ],
        out_specs=[
            pl.BlockSpec((_RB, 256), lambda g: (g, 0)),
            pl.BlockSpec((2, 256), lambda g: (0, 0)),
        ],
        out_shape=[
            jax.ShapeDtypeStruct((_A, 256), jnp.float32),
            jax.ShapeDtypeStruct((2, 256), jnp.float32),
        ],
    )(x, y, x2, y2, W1)

    h1 = jnp.concatenate([h1a, h1b], axis=0)
    st1 = sta + stb

    h2, st2 = pl.pallas_call(
        functools.partial(_k2_body, n_pos=n_pos),
        grid=(M // _PB,),
        in_specs=[
            pl.BlockSpec((_PB, 256), lambda i: (i, 0)),
            pl.BlockSpec((2, 256), lambda i: (0, 0)),
            pl.BlockSpec((1, 256), lambda i: (0, 0)),
            pl.BlockSpec((1, 256), lambda i: (0, 0)),
            pl.BlockSpec((128, 256), lambda i: (0, 0)),
        ],
        out_specs=[
            pl.BlockSpec((_PB, 128), lambda i: (i, 0)),
            pl.BlockSpec((2, 128), lambda i: (0, 0)),
        ],
        out_shape=[
            jax.ShapeDtypeStruct((M, 128), jnp.float32),
            jax.ShapeDtypeStruct((2, 128), jnp.float32),
        ],
    )(h1, st1, g1.reshape(1, 256), b1.reshape(1, 256), W2)

    out = pl.pallas_call(
        functools.partial(_k3_body, n_pos=n_pos),
        grid=(M // _PB,),
        in_specs=[
            pl.BlockSpec((_PB, 128), lambda i: (i, 0)),
            pl.BlockSpec((2, 128), lambda i: (0, 0)),
            pl.BlockSpec((1, 128), lambda i: (0, 0)),
            pl.BlockSpec((1, 128), lambda i: (0, 0)),
            pl.BlockSpec((1, 128), lambda i: (0, 0)),
        ],
        out_specs=pl.BlockSpec((_PB, 1), lambda i: (i, 0)),
        out_shape=jax.ShapeDtypeStruct((M, 1), jnp.float32),
    )(h2, st2, g2.reshape(1, 128), b2.reshape(1, 128), W3)

    return out.reshape(B, N, 1)


# speedup vs baseline: 2.0975x; 1.0549x over previous
import functools

import jax
import jax.numpy as jnp
from jax.experimental import pallas as pl
from jax.experimental.pallas import tpu as pltpu
from jax.experimental.pallas import tpu_sc as plsc

_K = 20
_EPS = 1e-5
_RB = 256
_PB = 2048
_NEG = float("-inf")

_NC, _NS, _L = 2, 16, 16
_W = _NC * _NS
_CH = 8
_CAP = 4096 + 16


def _score_body(xf_ref, yf_ref, xr_ref, s_ref):
    xf = xf_ref[0]
    colb = -jnp.sum(xf * xf, axis=1, keepdims=True)
    inner = jax.lax.dot_general(xr_ref[...], yf_ref[0], (((1,), (1,)), ((), ())),
                                preferred_element_type=jnp.float32)
    srow = jax.lax.dot_general(
        jnp.ones((_RB, 1), jnp.float32), colb, (((1,), (1,)), ((), ())),
        preferred_element_type=jnp.float32,
        precision=jax.lax.Precision.HIGHEST)
    s_ref[...] = 2.0 * inner + srow


def _tc_topk_body(xf_ref, yf_ref, xr_ref, yr_ref, w1_ref, h1_ref, st_ref):
    g = pl.program_id(0)
    xf = xf_ref[0]
    colb = -jnp.sum(xf * xf, axis=1, keepdims=True)
    inner = jax.lax.dot_general(xr_ref[...], yf_ref[0], (((1,), (1,)), ((), ())),
                                preferred_element_type=jnp.float32)
    srow = jax.lax.dot_general(
        jnp.ones((_RB, 1), jnp.float32), colb, (((1,), (1,)), ((), ())),
        preferred_element_type=jnp.float32,
        precision=jax.lax.Precision.HIGHEST)
    s = 2.0 * inner + srow

    out = jnp.full((_RB, 32), _NEG, jnp.float32)
    lane = jax.lax.broadcasted_iota(jnp.int32, (_RB, 32), 1)
    for j in range(_K):
        m = jnp.max(s, axis=1, keepdims=True)
        s = jnp.where(s == m, _NEG, s)
        out = jnp.where(lane == j, m, out)

    rowb = -jnp.sum(yr_ref[...] * yr_ref[...], axis=1, keepdims=True)
    feat = out[:, :_K] + rowb
    h1 = jax.lax.dot_general(feat, w1_ref[...], (((1,), (1,)), ((), ())),
                             preferred_element_type=jnp.float32)
    h1_ref[...] = h1
    s1 = jnp.sum(h1, axis=0, keepdims=True)
    s2 = jnp.sum(h1 * h1, axis=0, keepdims=True)
    st = jnp.concatenate([s1, s2], axis=0)

    @pl.when(g == 0)
    def _():
        st_ref[...] = st

    @pl.when(g != 0)
    def _():
        st_ref[...] = st_ref[...] + st


def _sc_row(buf, r, cand, outb):
    neg = jnp.full((_L,), _NEG, jnp.float32)
    iota = jax.lax.iota(jnp.int32, _L)

    def p1(k, carry):
        m0, m1 = carry
        v0 = buf[r, pl.ds(2 * k * _L, _L)]
        v1 = buf[r, pl.ds((2 * k + 1) * _L, _L)]
        return jnp.maximum(m0, v0), jnp.maximum(m1, v1)

    m0, m1 = plsc.parallel_loop(0, 4096 // (2 * _L), unroll=8,
                                carry=(neg, neg))(p1)

    t = jnp.minimum(jnp.min(m0), jnp.min(m1))

    ones = jnp.ones((_L,), jnp.int32)

    def p2(k, cnt):
        v = buf[r, pl.ds(k * _L, _L)]
        msk = v >= t
        pos = plsc.cumsum(ones, mask=msk)
        plsc.store_scatter(cand, [cnt + pos - 1], v, mask=msk)
        return cnt + plsc.all_reduce_population_count(msk)

    cntv = plsc.parallel_loop(0, 4096 // _L, unroll=8,
                              carry=jnp.zeros((_L,), jnp.int32))(p2)
    c = jnp.max(cntv)

    def fast():
        vs = []
        for k in range(8):
            v = cand[pl.ds(k * _L, _L)]
            vs.append(jnp.where(k * _L + iota < c, v, neg))

        def rnd(j, carry):
            o0, o1 = carry[0], carry[1]
            w = list(carry[2:])
            t1 = jnp.maximum(jnp.maximum(w[0], w[1]), jnp.maximum(w[2], w[3]))
            t2 = jnp.maximum(jnp.maximum(w[4], w[5]), jnp.maximum(w[6], w[7]))
            m = jnp.max(jnp.maximum(t1, t2))
            w = [jnp.where(v == m, _NEG, v) for v in w]
            o0 = jnp.where(iota == j, m, o0)
            o1 = jnp.where(iota == j - _L, m, o1)
            return (o0, o1, *w)

        res = jax.lax.fori_loop(0, _K, rnd, (neg, neg, *vs))
        return res[0], res[1]

    def slow():
        plsc.store_scatter(cand, [c + iota], neg)
        nv = c // _L + 1

        def p3(j, carry):
            o0, o1 = carry

            def emax(k, mm):
                return jnp.maximum(mm, cand[pl.ds(k * _L, _L)])

            m = jnp.max(jax.lax.fori_loop(0, nv, emax, neg))

            def erm(k, z):
                v = cand[pl.ds(k * _L, _L)]
                cand[pl.ds(k * _L, _L)] = jnp.where(v == m, _NEG, v)
                return z

            jax.lax.fori_loop(0, nv, erm, 0)
            o0 = jnp.where(iota == j, m, o0)
            o1 = jnp.where(iota == j - _L, m, o1)
            return o0, o1

        return jax.lax.fori_loop(0, _K, p3, (neg, neg))

    o0, o1 = jax.lax.cond(c <= 8 * _L, fast, slow)
    outb[r, pl.ds(0, _L)] = o0
    outb[r, pl.ds(_L, _L)] = o1


def _sc_topk(s2d, n_rows):
    rpw = n_rows // _W
    nch = rpw // _CH
    mesh = plsc.VectorSubcoreMesh(core_axis_name="c", subcore_axis_name="s")

    @functools.partial(
        pl.kernel,
        out_type=jax.ShapeDtypeStruct((n_rows, 2 * _L), jnp.float32),
        mesh=mesh,
        scratch_types=[
            pltpu.VMEM((_CH, 4096), jnp.float32),
            pltpu.VMEM((_CH, 4096), jnp.float32),
            pltpu.VMEM((_CAP,), jnp.float32),
            pltpu.VMEM((_CH, 2 * _L), jnp.float32),
            pltpu.SemaphoreType.DMA,
            pltpu.SemaphoreType.DMA,
        ],
        compiler_params=pltpu.CompilerParams(needs_layout_passes=False),
    )
    def topk_kernel(s_hbm, o_hbm, buf0, buf1, cand, outb, sem0, sem1):
        wid = jax.lax.axis_index("s") * _NC + jax.lax.axis_index("c")
        base = wid * rpw
        pltpu.async_copy(s_hbm.at[pl.ds(base, _CH)], buf0, sem0)

        def do_chunk(ci, buf, sem):
            pltpu.make_async_copy(
                s_hbm.at[pl.ds(base + ci * _CH, _CH)], buf, sem).wait()
            for r in range(_CH):
                _sc_row(buf, r, cand, outb)
            pltpu.sync_copy(outb, o_hbm.at[pl.ds(base + ci * _CH, _CH)])

        def pair(g, z):
            c0 = 2 * g
            pltpu.async_copy(
                s_hbm.at[pl.ds(base + (c0 + 1) * _CH, _CH)], buf1, sem1)
            do_chunk(c0, buf0, sem0)

            @pl.when(c0 + 2 < nch)
            def _():
                pltpu.async_copy(
                    s_hbm.at[pl.ds(base + (c0 + 2) * _CH, _CH)], buf0, sem0)

            do_chunk(c0 + 1, buf1, sem1)
            return z

        jax.lax.fori_loop(0, nch // 2, pair, 0)

    return topk_kernel(s2d)


def _h1_body(f_ref, y_ref, w1_ref, h1_ref, st_ref):
    rowb = -jnp.sum(y_ref[...] * y_ref[...], axis=1, keepdims=True)
    feat = f_ref[:, :_K] + rowb
    h1 = jax.lax.dot_general(feat, w1_ref[...], (((1,), (1,)), ((), ())),
                             preferred_element_type=jnp.float32)
    h1_ref[...] = h1
    s1 = jnp.sum(h1, axis=0, keepdims=True)
    s2 = jnp.sum(h1 * h1, axis=0, keepdims=True)
    st = jnp.concatenate([s1, s2], axis=0)

    @pl.when(pl.program_id(0) == 0)
    def _():
        st_ref[...] = st

    @pl.when(pl.program_id(0) != 0)
    def _():
        st_ref[...] = st_ref[...] + st


def _bn_relu_mm(h_ref, st_ref, g_ref, b_ref, w_ref, n_pos):
    s1 = st_ref[0:1, :]
    s2 = st_ref[1:2, :]
    mean = s1 / n_pos
    var = s2 / n_pos - mean * mean
    scale = g_ref[...] * jax.lax.rsqrt(var + _EPS)
    shift = b_ref[...] - mean * scale
    a = jnp.maximum(h_ref[...] * scale + shift, 0.0)
    return jax.lax.dot_general(a, w_ref[...], (((1,), (1,)), ((), ())),
                               preferred_element_type=jnp.float32)


def _k2_body(h1_ref, st_ref, g_ref, b_ref, w2_ref, h2_ref, st2_ref, *, n_pos):
    h2 = _bn_relu_mm(h1_ref, st_ref, g_ref, b_ref, w2_ref, n_pos)
    h2_ref[...] = h2
    s1 = jnp.sum(h2, axis=0, keepdims=True)
    s2 = jnp.sum(h2 * h2, axis=0, keepdims=True)
    st = jnp.concatenate([s1, s2], axis=0)

    @pl.when(pl.program_id(0) == 0)
    def _():
        st2_ref[...] = st

    @pl.when(pl.program_id(0) != 0)
    def _():
        st2_ref[...] = st2_ref[...] + st


def _k3_body(h2_ref, st_ref, g_ref, b_ref, w3_ref, o_ref, *, n_pos):
    h3 = _bn_relu_mm(h2_ref, st_ref, g_ref, b_ref, w3_ref, n_pos)
    o_ref[...] = jax.nn.sigmoid(h3)


_A = 16384


@jax.jit
def kernel(x, y, W1, W2, W3, g1, b1, g2, b2):
    B, N, C = x.shape
    M = B * N
    n_pos = float(M)
    x2 = x.reshape(M, C)
    y2 = y.reshape(M, C)
    nblk = N // _RB
    a_blk = _A // _RB
    a_pb = _A // _PB

    s2d = pl.pallas_call(
        _score_body,
        grid=((M - _A) // _RB,),
        in_specs=[
            pl.BlockSpec((1, N, C), lambda g: ((a_blk + g) // nblk, 0, 0)),
            pl.BlockSpec((1, N, C), lambda g: ((a_blk + g) // nblk, 0, 0)),
            pl.BlockSpec((_RB, C), lambda g: (a_blk + g, 0)),
        ],
        out_specs=pl.BlockSpec((_RB, N), lambda g: (g, 0)),
        out_shape=jax.ShapeDtypeStruct((M - _A, N), jnp.float32),
    )(x, y, x2)

    feat = _sc_topk(s2d, M - _A)

    h1b, stb = pl.pallas_call(
        _h1_body,
        grid=((M - _A) // _PB,),
        in_specs=[
            pl.BlockSpec((_PB, 2 * _L), lambda i: (i, 0)),
            pl.BlockSpec((_PB, C), lambda i: (a_pb + i, 0)),
            pl.BlockSpec((256, _K), lambda i: (0, 0)),
        ],
        out_specs=[
            pl.BlockSpec((_PB, 256), lambda i: (i, 0)),
            pl.BlockSpec((2, 256), lambda i: (0, 0)),
        ],
        out_shape=[
            jax.ShapeDtypeStruct((M - _A, 256), jnp.float32),
            jax.ShapeDtypeStruct((2, 256), jnp.float32),
        ],
    )(feat, y2, W1)

    h1a, sta = pl.pallas_call(
        _tc_topk_body,
        grid=(a_blk,),
        in_specs=[
            pl.BlockSpec((1, N, C), lambda g: (g // nblk, 0, 0)),
            pl.BlockSpec((1, N, C), lambda g: (g // nblk, 0, 0)),
            pl.BlockSpec((_RB, C), lambda g: (g, 0)),
            pl.BlockSpec((_RB, C), lambda g: (g, 0)),
            pl.BlockSpec((256, _K), lambda g: (0, 0)),
        ],
        out_specs=[
            pl.BlockSpec((_RB, 256), lambda g: (g, 0)),
            pl.BlockSpec((2, 256), lambda g: (0, 0)),
        ],
        out_shape=[
            jax.ShapeDtypeStruct((_A, 256), jnp.float32),
            jax.ShapeDtypeStruct((2, 256), jnp.float32),
        ],
    )(x, y, x2, y2, W1)

    h1 = jnp.concatenate([h1a, h1b], axis=0)
    st1 = sta + stb

    h2, st2 = pl.pallas_call(
        functools.partial(_k2_body, n_pos=n_pos),
        grid=(M // _PB,),
        in_specs=[
            pl.BlockSpec((_PB, 256), lambda i: (i, 0)),
            pl.BlockSpec((2, 256), lambda i: (0, 0)),
            pl.BlockSpec((1, 256), lambda i: (0, 0)),
            pl.BlockSpec((1, 256), lambda i: (0, 0)),
            pl.BlockSpec((128, 256), lambda i: (0, 0)),
        ],
        out_specs=[
            pl.BlockSpec((_PB, 128), lambda i: (i, 0)),
            pl.BlockSpec((2, 128), lambda i: (0, 0)),
        ],
        out_shape=[
            jax.ShapeDtypeStruct((M, 128), jnp.float32),
            jax.ShapeDtypeStruct((2, 128), jnp.float32),
        ],
    )(h1, st1, g1.reshape(1, 256), b1.reshape(1, 256), W2)

    out = pl.pallas_call(
        functools.partial(_k3_body, n_pos=n_pos),
        grid=(M // _PB,),
        in_specs=[
            pl.BlockSpec((_PB, 128), lambda i: (i, 0)),
            pl.BlockSpec((2, 128), lambda i: (0, 0)),
            pl.BlockSpec((1, 128), lambda i: (0, 0)),
            pl.BlockSpec((1, 128), lambda i: (0, 0)),
            pl.BlockSpec((1, 128), lambda i: (0, 0)),
        ],
        out_specs=pl.BlockSpec((_PB, 1), lambda i: (i, 0)),
        out_shape=jax.ShapeDtypeStruct((M, 1), jnp.float32),
    )(h2, st2, g2.reshape(1, 128), b2.reshape(1, 128), W3)

    return out.reshape(B, N, 1)
